# baseline (speedup 1.0000x reference)
import functools

import jax
import jax.numpy as jnp
from jax import lax
from jax.experimental import pallas as pl
from jax.experimental.pallas import tpu as pltpu
from jax.experimental.pallas import tpu_sc as plsc

_N = 10000
_E = 320000
_D = 128

_NC = 2
_NS = 16
_NW = _NC * _NS
_EPW = _E // _NW
_C = 80
_NCH = _EPW // _C
_RPT = _N // _NS

_ZR = 125


def _edge_body(ft_hbm, el_hbm, er_hbm, idx_hbm,
               acc_out, den_out,
               idxb, el_v, er_v, rows_v, w_v, zden,
               acc_sh, den_sh, sem_g, sem_s):
    cid = lax.axis_index("c")
    sid = lax.axis_index("s")
    wid = cid * _NS + sid

    pltpu.sync_copy(el_hbm, el_v)
    pltpu.sync_copy(er_hbm, er_v)

    zv = jnp.zeros((16,), jnp.float32)

    def zrow(i, _):
        for k in range(_D // 16):
            rows_v[0, i, pl.ds(k * 16, 16)] = zv
        return _
    lax.fori_loop(0, _C, zrow, None)

    def zden_row(i, _):
        zden[pl.ds(i * 16, 16)] = zv
        return _
    lax.fori_loop(0, 40, zden_row, None)

    for j in range(7):
        pltpu.sync_copy(rows_v.at[0],
                        acc_sh.at[pl.ds(sid * _RPT + j * _C, _C)])
    pltpu.sync_copy(rows_v.at[0, pl.ds(0, _RPT - 7 * _C)],
                    acc_sh.at[pl.ds(sid * _RPT + 7 * _C, _RPT - 7 * _C)])

    @pl.when(sid < _NS - 1)
    def _():
        pltpu.sync_copy(zden.at[pl.ds(0, 624)],
                        den_sh.at[pl.ds(sid * 624, 624)])

    @pl.when(sid == _NS - 1)
    def _():
        pltpu.sync_copy(zden, den_sh.at[pl.ds(624 * (_NS - 1), 640)])

    plsc.subcore_barrier()

    pltpu.sync_copy(idx_hbm.at[wid, 0], idxb.at[0])
    pltpu.async_copy(ft_hbm.at[idxb.at[0, 0]], rows_v.at[0], sem_g)

    def chunk(ci, _):
        b = lax.rem(ci, 2)
        nb = 1 - b
        not_last = ci < _NCH - 1

        def grp(gi, _):
            s16 = idxb[b, 0, pl.ds(gi * 16, 16)]
            d16 = idxb[b, 1, pl.ds(gi * 16, 16)]
            e = plsc.load_gather(el_v, [s16]) + plsc.load_gather(er_v, [d16])
            e = jnp.where(e >= 0.0, e, e * 0.2)
            w_v[b, pl.ds(gi * 16, 16)] = jnp.exp(e)
            return _
        lax.fori_loop(0, _C // 16, grp, None)

        @pl.when(ci > 0)
        def _():
            pltpu.make_async_copy(ft_hbm.at[pl.ds(0, _C)], rows_v.at[nb],
                                  sem_s).wait()
            pltpu.make_async_copy(el_hbm.at[pl.ds(0, _C)], w_v.at[nb],
                                  sem_s).wait()

        @pl.when(not_last)
        def _():
            pltpu.sync_copy(idx_hbm.at[wid, ci + 1], idxb.at[nb])
            pltpu.async_copy(ft_hbm.at[idxb.at[nb, 0]], rows_v.at[nb], sem_g)

        pltpu.make_async_copy(ft_hbm.at[pl.ds(0, _C)], rows_v.at[b],
                              sem_g).wait()

        def scale(gi, _):
            for j in range(16):
                ei = gi * 16 + j
                wb = plsc.load_gather(w_v.at[b],
                                      [jnp.full((16,), ei, jnp.int32)])
                for k in range(_D // 16):
                    rows_v[b, ei, pl.ds(k * 16, 16)] = (
                        rows_v[b, ei, pl.ds(k * 16, 16)] * wb)
            return _
        lax.fori_loop(0, _C // 16, scale, None)

        pltpu.async_copy(rows_v.at[b], acc_sh.at[idxb.at[b, 1]], sem_s,
                         add=True)
        pltpu.async_copy(w_v.at[b], den_sh.at[idxb.at[b, 1]], sem_s,
                         add=True)
        return _
    lax.fori_loop(0, _NCH, chunk, None)

    pltpu.make_async_copy(ft_hbm.at[pl.ds(0, _C)], rows_v.at[0], sem_s).wait()
    pltpu.make_async_copy(el_hbm.at[pl.ds(0, _C)], w_v.at[0], sem_s).wait()

    plsc.subcore_barrier()

    pltpu.sync_copy(acc_sh.at[pl.ds(sid * _RPT, _RPT)],
                    acc_out.at[cid, pl.ds(sid * _RPT, _RPT)])

    @pl.when(sid == 0)
    def _():
        pltpu.sync_copy(den_sh, den_out.at[cid])


def _edge_call(ft, el, er, idx):
    mesh = plsc.VectorSubcoreMesh(core_axis_name="c", subcore_axis_name="s",
                                  num_cores=_NC, num_subcores=_NS)
    f = pl.kernel(
        _edge_body,
        out_type=(jax.ShapeDtypeStruct((_NC, _N, _D), jnp.float32),
                  jax.ShapeDtypeStruct((_NC, _N), jnp.float32)),
        mesh=mesh,
        scratch_types=[
            pltpu.VMEM((2, 2, _C), jnp.int32),
            pltpu.VMEM((_N,), jnp.float32),
            pltpu.VMEM((_N,), jnp.float32),
            pltpu.VMEM((2, _C, _D), jnp.float32),
            pltpu.VMEM((2, _C), jnp.float32),
            pltpu.VMEM((640,), jnp.float32),
            pltpu.VMEM_SHARED((_N, _D), jnp.float32),
            pltpu.VMEM_SHARED((_N,), jnp.float32),
            pltpu.SemaphoreType.DMA,
            pltpu.SemaphoreType.DMA,
        ],
        compiler_params=pltpu.CompilerParams(use_tc_tiling_on_sc=False,
                                             needs_layout_passes=False),
    )
    return f(ft, el, er, idx)


_B = 2000


def _prep1_body(x_ref, w_ref, al_ref, ar_ref, ft_ref, el_ref, er_ref):
    ft = jnp.dot(x_ref[...], w_ref[...], preferred_element_type=jnp.float32)
    ft_ref[...] = ft
    el_ref[...] = jnp.sum(ft * al_ref[...], axis=1, keepdims=True)
    er_ref[...] = jnp.sum(ft * ar_ref[...], axis=1, keepdims=True)


def _prep1(x, W, al, ar):
    return pl.pallas_call(
        _prep1_body,
        grid=(_N // _B,),
        in_specs=[
            pl.BlockSpec((_B, _D), lambda i: (i, 0)),
            pl.BlockSpec((_D, _D), lambda i: (0, 0)),
            pl.BlockSpec((1, _D), lambda i: (0, 0)),
            pl.BlockSpec((1, _D), lambda i: (0, 0)),
        ],
        out_specs=[
            pl.BlockSpec((_B, _D), lambda i: (i, 0)),
            pl.BlockSpec((_B, 1), lambda i: (i, 0)),
            pl.BlockSpec((_B, 1), lambda i: (i, 0)),
        ],
        out_shape=[
            jax.ShapeDtypeStruct((_N, _D), jnp.float32),
            jax.ShapeDtypeStruct((_N, 1), jnp.float32),
            jax.ShapeDtypeStruct((_N, 1), jnp.float32),
        ],
    )(x, W, al, ar)


def _prep2_body(acc_ref, den_ref, b_ref, w_ref, al_ref, ar_ref,
                ft_ref, el_ref, er_ref):
    a = acc_ref[0] + acc_ref[1]
    dn = den_ref[0] + den_ref[1]
    h = a / (dn + 1e-9) + b_ref[...]
    ft = jnp.dot(h, w_ref[...], preferred_element_type=jnp.float32)
    ft_ref[...] = ft
    el_ref[...] = jnp.sum(ft * al_ref[...], axis=1, keepdims=True)
    er_ref[...] = jnp.sum(ft * ar_ref[...], axis=1, keepdims=True)


def _prep2(acc, den, b, W, al, ar):
    den = den.reshape(2, _N, 1)
    return pl.pallas_call(
        _prep2_body,
        grid=(_N // _B,),
        in_specs=[
            pl.BlockSpec((2, _B, _D), lambda i: (0, i, 0)),
            pl.BlockSpec((2, _B, 1), lambda i: (0, i, 0)),
            pl.BlockSpec((1, _D), lambda i: (0, 0)),
            pl.BlockSpec((_D, _D), lambda i: (0, 0)),
            pl.BlockSpec((1, _D), lambda i: (0, 0)),
            pl.BlockSpec((1, _D), lambda i: (0, 0)),
        ],
        out_specs=[
            pl.BlockSpec((_B, _D), lambda i: (i, 0)),
            pl.BlockSpec((_B, 1), lambda i: (i, 0)),
            pl.BlockSpec((_B, 1), lambda i: (i, 0)),
        ],
        out_shape=[
            jax.ShapeDtypeStruct((_N, _D), jnp.float32),
            jax.ShapeDtypeStruct((_N, 1), jnp.float32),
            jax.ShapeDtypeStruct((_N, 1), jnp.float32),
        ],
    )(acc, den, b, W, al, ar)


def _fin_body(acc_ref, den_ref, b_ref, out_ref):
    a = acc_ref[0] + acc_ref[1]
    dn = den_ref[0] + den_ref[1]
    out_ref[...] = a / (dn + 1e-9) + b_ref[...]


def _fin(acc, den, b):
    den = den.reshape(2, _N, 1)
    return pl.pallas_call(
        _fin_body,
        grid=(_N // _B,),
        in_specs=[
            pl.BlockSpec((2, _B, _D), lambda i: (0, i, 0)),
            pl.BlockSpec((2, _B, 1), lambda i: (0, i, 0)),
            pl.BlockSpec((1, _D), lambda i: (0, 0)),
        ],
        out_specs=pl.BlockSpec((_B, _D), lambda i: (i, 0)),
        out_shape=jax.ShapeDtypeStruct((_N, _D), jnp.float32),
    )(acc, den, b)


def kernel(g, in_feat, W1, al1, ar1, b1, W2, al2, ar2, b2):
    g = g.astype(jnp.int32)
    idx = g.reshape(2, _NW, _NCH, _C).transpose(1, 2, 0, 3)
    b1r = b1.reshape(1, _D)
    b2r = b2.reshape(1, _D)

    ft1, el1, er1 = _prep1(in_feat, W1, al1, ar1)
    acc1, den1 = _edge_call(ft1, el1.reshape(_N), er1.reshape(_N), idx)
    ft2, el2, er2 = _prep2(acc1, den1, b1r, W2, al2, ar2)
    acc2, den2 = _edge_call(ft2, el2.reshape(_N), er2.reshape(_N), idx)
    out = _fin(acc2, den2, b2r)
    return out.reshape(_N, 1, _D)


# speedup vs baseline: 1.9550x; 1.9550x over previous
import functools

import jax
import jax.numpy as jnp
from jax import lax
from jax.experimental import pallas as pl
from jax.experimental.pallas import tpu as pltpu
from jax.experimental.pallas import tpu_sc as plsc

_N = 10000
_E = 320000
_D = 128

_NC = 2
_NS = 16
_NW = _NC * _NS
_EPW = _E // _NW
_C = 80
_NCH = _EPW // _C
_RPT = _N // _NS

_ZR = 125


def _edge_body(ft_hbm, el_hbm, er_hbm, idx_hbm,
               acc_out, den_out,
               idxb, el_v, er_v, rows_v, w_v, zden,
               acc_sh, den_sh, sem_g, sem_s):
    cid = lax.axis_index("c")
    sid = lax.axis_index("s")
    wid = cid * _NS + sid

    pltpu.sync_copy(el_hbm, el_v)
    pltpu.sync_copy(er_hbm, er_v)

    zv = jnp.zeros((16,), jnp.float32)

    def zrow(i, _):
        for k in range(_D // 16):
            rows_v[0, i, pl.ds(k * 16, 16)] = zv
        return _
    lax.fori_loop(0, _C, zrow, None)

    def zden_row(i, _):
        zden[pl.ds(i * 16, 16)] = zv
        return _
    lax.fori_loop(0, 40, zden_row, None)

    for j in range(7):
        pltpu.sync_copy(rows_v.at[0],
                        acc_sh.at[pl.ds(sid * _RPT + j * _C, _C)])
    pltpu.sync_copy(rows_v.at[0, pl.ds(0, _RPT - 7 * _C)],
                    acc_sh.at[pl.ds(sid * _RPT + 7 * _C, _RPT - 7 * _C)])

    @pl.when(sid < _NS - 1)
    def _():
        pltpu.sync_copy(zden.at[pl.ds(0, 624)],
                        den_sh.at[pl.ds(sid * 624, 624)])

    @pl.when(sid == _NS - 1)
    def _():
        pltpu.sync_copy(zden, den_sh.at[pl.ds(624 * (_NS - 1), 640)])

    plsc.subcore_barrier()

    def half(ci, b, drain):
        nb = 1 - b

        def grp(gi, _):
            s16 = idxb[b, 0, pl.ds(gi * 16, 16)]
            d16 = idxb[b, 1, pl.ds(gi * 16, 16)]
            e = plsc.load_gather(el_v, [s16]) + plsc.load_gather(er_v, [d16])
            e = jnp.where(e >= 0.0, e, e * 0.2)
            w_v[b, pl.ds(gi * 16, 16)] = jnp.exp(e)
            return _
        lax.fori_loop(0, _C // 16, grp, None)

        if drain:
            pltpu.make_async_copy(ft_hbm.at[pl.ds(0, _C)], rows_v.at[nb],
                                  sem_s).wait()
            pltpu.make_async_copy(el_hbm.at[pl.ds(0, _C)], w_v.at[nb],
                                  sem_s).wait()

        @pl.when(ci < _NCH - 1)
        def _():
            pltpu.sync_copy(idx_hbm.at[wid, ci + 1], idxb.at[nb])
            pltpu.async_copy(ft_hbm.at[idxb.at[nb, 0]], rows_v.at[nb], sem_g)

        pltpu.make_async_copy(ft_hbm.at[pl.ds(0, _C)], rows_v.at[b],
                              sem_g).wait()

        def scale(gi, _):
            for j in range(16):
                ei = gi * 16 + j
                wb = plsc.load_gather(w_v.at[b],
                                      [jnp.full((16,), ei, jnp.int32)])
                for k in range(_D // 16):
                    rows_v[b, ei, pl.ds(k * 16, 16)] = (
                        rows_v[b, ei, pl.ds(k * 16, 16)] * wb)
            return _
        lax.fori_loop(0, _C // 16, scale, None)

        pltpu.async_copy(rows_v.at[b], acc_sh.at[idxb.at[b, 1]], sem_s,
                         add=True)
        pltpu.async_copy(w_v.at[b], den_sh.at[idxb.at[b, 1]], sem_s,
                         add=True)

    pltpu.sync_copy(idx_hbm.at[wid, 0], idxb.at[0])
    pltpu.async_copy(ft_hbm.at[idxb.at[0, 0]], rows_v.at[0], sem_g)
    half(0, 0, drain=False)

    def pair(i, _):
        half(1 + 2 * i, 1, drain=True)
        half(2 + 2 * i, 0, drain=True)
        return _
    lax.fori_loop(0, (_NCH - 1) // 2, pair, None)

    pltpu.make_async_copy(ft_hbm.at[pl.ds(0, _C)], rows_v.at[0], sem_s).wait()
    pltpu.make_async_copy(el_hbm.at[pl.ds(0, _C)], w_v.at[0], sem_s).wait()

    plsc.subcore_barrier()

    pltpu.sync_copy(acc_sh.at[pl.ds(sid * _RPT, _RPT)],
                    acc_out.at[cid, pl.ds(sid * _RPT, _RPT)])

    @pl.when(sid == 0)
    def _():
        pltpu.sync_copy(den_sh, den_out.at[cid])


def _edge_call(ft, el, er, idx):
    mesh = plsc.VectorSubcoreMesh(core_axis_name="c", subcore_axis_name="s",
                                  num_cores=_NC, num_subcores=_NS)
    f = pl.kernel(
        _edge_body,
        out_type=(jax.ShapeDtypeStruct((_NC, _N, _D), jnp.float32),
                  jax.ShapeDtypeStruct((_NC, _N), jnp.float32)),
        mesh=mesh,
        scratch_types=[
            pltpu.VMEM((2, 2, _C), jnp.int32),
            pltpu.VMEM((_N,), jnp.float32),
            pltpu.VMEM((_N,), jnp.float32),
            pltpu.VMEM((2, _C, _D), jnp.float32),
            pltpu.VMEM((2, _C), jnp.float32),
            pltpu.VMEM((640,), jnp.float32),
            pltpu.VMEM_SHARED((_N, _D), jnp.float32),
            pltpu.VMEM_SHARED((_N,), jnp.float32),
            pltpu.SemaphoreType.DMA,
            pltpu.SemaphoreType.DMA,
        ],
        compiler_params=pltpu.CompilerParams(use_tc_tiling_on_sc=False,
                                             needs_layout_passes=False),
    )
    return f(ft, el, er, idx)


_B = 2000


def _prep1_body(x_ref, w_ref, al_ref, ar_ref, ft_ref, el_ref, er_ref):
    ft = jnp.dot(x_ref[...], w_ref[...], preferred_element_type=jnp.float32)
    ft_ref[...] = ft
    el_ref[...] = jnp.sum(ft * al_ref[...], axis=1, keepdims=True)
    er_ref[...] = jnp.sum(ft * ar_ref[...], axis=1, keepdims=True)


def _prep1(x, W, al, ar):
    return pl.pallas_call(
        _prep1_body,
        grid=(_N // _B,),
        in_specs=[
            pl.BlockSpec((_B, _D), lambda i: (i, 0)),
            pl.BlockSpec((_D, _D), lambda i: (0, 0)),
            pl.BlockSpec((1, _D), lambda i: (0, 0)),
            pl.BlockSpec((1, _D), lambda i: (0, 0)),
        ],
        out_specs=[
            pl.BlockSpec((_B, _D), lambda i: (i, 0)),
            pl.BlockSpec((_B, 1), lambda i: (i, 0)),
            pl.BlockSpec((_B, 1), lambda i: (i, 0)),
        ],
        out_shape=[
            jax.ShapeDtypeStruct((_N, _D), jnp.float32),
            jax.ShapeDtypeStruct((_N, 1), jnp.float32),
            jax.ShapeDtypeStruct((_N, 1), jnp.float32),
        ],
    )(x, W, al, ar)


def _prep2_body(acc_ref, den_ref, b_ref, w_ref, al_ref, ar_ref,
                ft_ref, el_ref, er_ref):
    a = acc_ref[0] + acc_ref[1]
    dn = den_ref[0] + den_ref[1]
    h = a / (dn + 1e-9) + b_ref[...]
    ft = jnp.dot(h, w_ref[...], preferred_element_type=jnp.float32)
    ft_ref[...] = ft
    el_ref[...] = jnp.sum(ft * al_ref[...], axis=1, keepdims=True)
    er_ref[...] = jnp.sum(ft * ar_ref[...], axis=1, keepdims=True)


def _prep2(acc, den, b, W, al, ar):
    den = den.reshape(2, _N, 1)
    return pl.pallas_call(
        _prep2_body,
        grid=(_N // _B,),
        in_specs=[
            pl.BlockSpec((2, _B, _D), lambda i: (0, i, 0)),
            pl.BlockSpec((2, _B, 1), lambda i: (0, i, 0)),
            pl.BlockSpec((1, _D), lambda i: (0, 0)),
            pl.BlockSpec((_D, _D), lambda i: (0, 0)),
            pl.BlockSpec((1, _D), lambda i: (0, 0)),
            pl.BlockSpec((1, _D), lambda i: (0, 0)),
        ],
        out_specs=[
            pl.BlockSpec((_B, _D), lambda i: (i, 0)),
            pl.BlockSpec((_B, 1), lambda i: (i, 0)),
            pl.BlockSpec((_B, 1), lambda i: (i, 0)),
        ],
        out_shape=[
            jax.ShapeDtypeStruct((_N, _D), jnp.float32),
            jax.ShapeDtypeStruct((_N, 1), jnp.float32),
            jax.ShapeDtypeStruct((_N, 1), jnp.float32),
        ],
    )(acc, den, b, W, al, ar)


def _fin_body(acc_ref, den_ref, b_ref, out_ref):
    a = acc_ref[0] + acc_ref[1]
    dn = den_ref[0] + den_ref[1]
    out_ref[...] = a / (dn + 1e-9) + b_ref[...]


def _fin(acc, den, b):
    den = den.reshape(2, _N, 1)
    return pl.pallas_call(
        _fin_body,
        grid=(_N // _B,),
        in_specs=[
            pl.BlockSpec((2, _B, _D), lambda i: (0, i, 0)),
            pl.BlockSpec((2, _B, 1), lambda i: (0, i, 0)),
            pl.BlockSpec((1, _D), lambda i: (0, 0)),
        ],
        out_specs=pl.BlockSpec((_B, _D), lambda i: (i, 0)),
        out_shape=jax.ShapeDtypeStruct((_N, _D), jnp.float32),
    )(acc, den, b)


def kernel(g, in_feat, W1, al1, ar1, b1, W2, al2, ar2, b2):
    g = g.astype(jnp.int32)
    idx = g.reshape(2, _NW, _NCH, _C).transpose(1, 2, 0, 3)
    b1r = b1.reshape(1, _D)
    b2r = b2.reshape(1, _D)

    ft1, el1, er1 = _prep1(in_feat, W1, al1, ar1)
    acc1, den1 = _edge_call(ft1, el1.reshape(_N), er1.reshape(_N), idx)
    ft2, el2, er2 = _prep2(acc1, den1, b1r, W2, al2, ar2)
    acc2, den2 = _edge_call(ft2, el2.reshape(_N), er2.reshape(_N), idx)
    out = _fin(acc2, den2, b2r)
    return out.reshape(_N, 1, _D)


# speedup vs baseline: 2.3482x; 1.2011x over previous
import functools

import jax
import jax.numpy as jnp
from jax import lax
from jax.experimental import pallas as pl
from jax.experimental.pallas import tpu as pltpu
from jax.experimental.pallas import tpu_sc as plsc

_N = 10000
_E = 320000
_D = 128

_NC = 2
_NS = 16
_NW = _NC * _NS
_EPW = _E // _NW
_C = 80
_NCH = _EPW // _C
_RPT = _N // _NS

_ZR = 125
_SUP = 5


def _edge_body(ft_hbm, el_hbm, er_hbm, idx_hbm,
               acc_out, den_out,
               idxb, el_v, er_v, rows_v, w_v, zden,
               acc_sh, den_sh, sem_g, sem_s, sem_i):
    cid = lax.axis_index("c")
    sid = lax.axis_index("s")
    wid = cid * _NS + sid

    pltpu.sync_copy(el_hbm, el_v)
    pltpu.sync_copy(er_hbm, er_v)

    zv = jnp.zeros((16,), jnp.float32)

    def zrow(i, _):
        for k in range(_D // 16):
            rows_v[0, i, pl.ds(k * 16, 16)] = zv
        return _
    lax.fori_loop(0, _C, zrow, None)

    def zden_row(i, _):
        zden[pl.ds(i * 16, 16)] = zv
        return _
    lax.fori_loop(0, 40, zden_row, None)

    for j in range(7):
        pltpu.sync_copy(rows_v.at[0],
                        acc_sh.at[pl.ds(sid * _RPT + j * _C, _C)])
    pltpu.sync_copy(rows_v.at[0, pl.ds(0, _RPT - 7 * _C)],
                    acc_sh.at[pl.ds(sid * _RPT + 7 * _C, _RPT - 7 * _C)])

    @pl.when(sid < _NS - 1)
    def _():
        pltpu.sync_copy(zden.at[pl.ds(0, 624)],
                        den_sh.at[pl.ds(sid * 624, 624)])

    @pl.when(sid == _NS - 1)
    def _():
        pltpu.sync_copy(zden, den_sh.at[pl.ds(624 * (_NS - 1), 640)])

    plsc.subcore_barrier()

    def chunk_work(ci, rb, sb, j, drain):
        nrb = 1 - rb

        def grp(gi, _):
            s16 = idxb[sb, j, 0, pl.ds(gi * 16, 16)]
            d16 = idxb[sb, j, 1, pl.ds(gi * 16, 16)]
            e = plsc.load_gather(el_v, [s16]) + plsc.load_gather(er_v, [d16])
            e = jnp.where(e >= 0.0, e, e * 0.2)
            w_v[rb, pl.ds(gi * 16, 16)] = jnp.exp(e)
            return _
        lax.fori_loop(0, _C // 16, grp, None)

        if drain:
            pltpu.make_async_copy(ft_hbm.at[pl.ds(0, _C)], rows_v.at[nrb],
                                  sem_s).wait()
            pltpu.make_async_copy(el_hbm.at[pl.ds(0, _C)], w_v.at[nrb],
                                  sem_s).wait()

        if j == 1:
            @pl.when(ci + 4 < _NCH)
            def _():
                pltpu.async_copy(idx_hbm.at[wid, pl.ds(ci + 4, _SUP)],
                                 idxb.at[1 - sb], sem_i)

        if j < _SUP - 1:
            pltpu.async_copy(ft_hbm.at[idxb.at[sb, j + 1, 0]],
                             rows_v.at[nrb], sem_g)
        else:
            @pl.when(ci + 1 < _NCH)
            def _():
                pltpu.make_async_copy(idx_hbm.at[wid, pl.ds(0, _SUP)],
                                      idxb.at[1 - sb], sem_i).wait()
                pltpu.async_copy(ft_hbm.at[idxb.at[1 - sb, 0, 0]],
                                 rows_v.at[nrb], sem_g)

        pltpu.make_async_copy(ft_hbm.at[pl.ds(0, _C)], rows_v.at[rb],
                              sem_g).wait()

        def scale(gi, _):
            for jj in range(16):
                ei = gi * 16 + jj
                wb = plsc.load_gather(w_v.at[rb],
                                      [jnp.full((16,), ei, jnp.int32)])
                for k in range(_D // 16):
                    rows_v[rb, ei, pl.ds(k * 16, 16)] = (
                        rows_v[rb, ei, pl.ds(k * 16, 16)] * wb)
            return _
        lax.fori_loop(0, _C // 16, scale, None)

        pltpu.async_copy(rows_v.at[rb], acc_sh.at[idxb.at[sb, j, 1]], sem_s,
                         add=True)
        pltpu.async_copy(w_v.at[rb], den_sh.at[idxb.at[sb, j, 1]], sem_s,
                         add=True)

    pltpu.sync_copy(idx_hbm.at[wid, pl.ds(0, _SUP)], idxb.at[0])
    pltpu.async_copy(ft_hbm.at[idxb.at[0, 0, 0]], rows_v.at[0], sem_g)

    for j in range(_SUP):
        chunk_work(j, j % 2, 0, j, drain=(j > 0))

    def pair(p, _):
        base0 = 5 * (1 + 2 * p)
        for j in range(_SUP):
            chunk_work(base0 + j, (1 + j) % 2, 1, j, drain=True)
        base1 = 5 * (2 + 2 * p)
        for j in range(_SUP):
            chunk_work(base1 + j, j % 2, 0, j, drain=True)
        return _
    lax.fori_loop(0, (_NCH // _SUP - 1) // 2, pair, None)

    pltpu.make_async_copy(ft_hbm.at[pl.ds(0, _C)], rows_v.at[0], sem_s).wait()
    pltpu.make_async_copy(el_hbm.at[pl.ds(0, _C)], w_v.at[0], sem_s).wait()

    plsc.subcore_barrier()

    pltpu.sync_copy(acc_sh.at[pl.ds(sid * _RPT, _RPT)],
                    acc_out.at[cid, pl.ds(sid * _RPT, _RPT)])

    @pl.when(sid == 0)
    def _():
        pltpu.sync_copy(den_sh, den_out.at[cid])


def _edge_call(ft, el, er, idx):
    mesh = plsc.VectorSubcoreMesh(core_axis_name="c", subcore_axis_name="s",
                                  num_cores=_NC, num_subcores=_NS)
    f = pl.kernel(
        _edge_body,
        out_type=(jax.ShapeDtypeStruct((_NC, _N, _D), jnp.float32),
                  jax.ShapeDtypeStruct((_NC, _N), jnp.float32)),
        mesh=mesh,
        scratch_types=[
            pltpu.VMEM((2, _SUP, 2, _C), jnp.int32),
            pltpu.VMEM((_N,), jnp.float32),
            pltpu.VMEM((_N,), jnp.float32),
            pltpu.VMEM((2, _C, _D), jnp.float32),
            pltpu.VMEM((2, _C), jnp.float32),
            pltpu.VMEM((640,), jnp.float32),
            pltpu.VMEM_SHARED((_N, _D), jnp.float32),
            pltpu.VMEM_SHARED((_N,), jnp.float32),
            pltpu.SemaphoreType.DMA,
            pltpu.SemaphoreType.DMA,
            pltpu.SemaphoreType.DMA,
        ],
        compiler_params=pltpu.CompilerParams(use_tc_tiling_on_sc=False,
                                             needs_layout_passes=False),
    )
    return f(ft, el, er, idx)


_B = 2000


def _prep1_body(x_ref, w_ref, al_ref, ar_ref, ft_ref, el_ref, er_ref):
    ft = jnp.dot(x_ref[...], w_ref[...], preferred_element_type=jnp.float32)
    ft_ref[...] = ft
    el_ref[...] = jnp.sum(ft * al_ref[...], axis=1, keepdims=True)
    er_ref[...] = jnp.sum(ft * ar_ref[...], axis=1, keepdims=True)


def _prep1(x, W, al, ar):
    return pl.pallas_call(
        _prep1_body,
        grid=(_N // _B,),
        in_specs=[
            pl.BlockSpec((_B, _D), lambda i: (i, 0)),
            pl.BlockSpec((_D, _D), lambda i: (0, 0)),
            pl.BlockSpec((1, _D), lambda i: (0, 0)),
            pl.BlockSpec((1, _D), lambda i: (0, 0)),
        ],
        out_specs=[
            pl.BlockSpec((_B, _D), lambda i: (i, 0)),
            pl.BlockSpec((_B, 1), lambda i: (i, 0)),
            pl.BlockSpec((_B, 1), lambda i: (i, 0)),
        ],
        out_shape=[
            jax.ShapeDtypeStruct((_N, _D), jnp.float32),
            jax.ShapeDtypeStruct((_N, 1), jnp.float32),
            jax.ShapeDtypeStruct((_N, 1), jnp.float32),
        ],
    )(x, W, al, ar)


def _prep2_body(acc_ref, den_ref, b_ref, w_ref, al_ref, ar_ref,
                ft_ref, el_ref, er_ref):
    a = acc_ref[0] + acc_ref[1]
    dn = den_ref[0] + den_ref[1]
    h = a / (dn + 1e-9) + b_ref[...]
    ft = jnp.dot(h, w_ref[...], preferred_element_type=jnp.float32)
    ft_ref[...] = ft
    el_ref[...] = jnp.sum(ft * al_ref[...], axis=1, keepdims=True)
    er_ref[...] = jnp.sum(ft * ar_ref[...], axis=1, keepdims=True)


def _prep2(acc, den, b, W, al, ar):
    den = den.reshape(2, _N, 1)
    return pl.pallas_call(
        _prep2_body,
        grid=(_N // _B,),
        in_specs=[
            pl.BlockSpec((2, _B, _D), lambda i: (0, i, 0)),
            pl.BlockSpec((2, _B, 1), lambda i: (0, i, 0)),
            pl.BlockSpec((1, _D), lambda i: (0, 0)),
            pl.BlockSpec((_D, _D), lambda i: (0, 0)),
            pl.BlockSpec((1, _D), lambda i: (0, 0)),
            pl.BlockSpec((1, _D), lambda i: (0, 0)),
        ],
        out_specs=[
            pl.BlockSpec((_B, _D), lambda i: (i, 0)),
            pl.BlockSpec((_B, 1), lambda i: (i, 0)),
            pl.BlockSpec((_B, 1), lambda i: (i, 0)),
        ],
        out_shape=[
            jax.ShapeDtypeStruct((_N, _D), jnp.float32),
            jax.ShapeDtypeStruct((_N, 1), jnp.float32),
            jax.ShapeDtypeStruct((_N, 1), jnp.float32),
        ],
    )(acc, den, b, W, al, ar)


def _fin_body(acc_ref, den_ref, b_ref, out_ref):
    a = acc_ref[0] + acc_ref[1]
    dn = den_ref[0] + den_ref[1]
    out_ref[...] = a / (dn + 1e-9) + b_ref[...]


def _fin(acc, den, b):
    den = den.reshape(2, _N, 1)
    return pl.pallas_call(
        _fin_body,
        grid=(_N // _B,),
        in_specs=[
            pl.BlockSpec((2, _B, _D), lambda i: (0, i, 0)),
            pl.BlockSpec((2, _B, 1), lambda i: (0, i, 0)),
            pl.BlockSpec((1, _D), lambda i: (0, 0)),
        ],
        out_specs=pl.BlockSpec((_B, _D), lambda i: (i, 0)),
        out_shape=jax.ShapeDtypeStruct((_N, _D), jnp.float32),
    )(acc, den, b)


def kernel(g, in_feat, W1, al1, ar1, b1, W2, al2, ar2, b2):
    g = g.astype(jnp.int32)
    idx = g.reshape(2, _NW, _NCH, _C).transpose(1, 2, 0, 3)
    b1r = b1.reshape(1, _D)
    b2r = b2.reshape(1, _D)

    ft1, el1, er1 = _prep1(in_feat, W1, al1, ar1)
    acc1, den1 = _edge_call(ft1, el1.reshape(_N), er1.reshape(_N), idx)
    ft2, el2, er2 = _prep2(acc1, den1, b1r, W2, al2, ar2)
    acc2, den2 = _edge_call(ft2, el2.reshape(_N), er2.reshape(_N), idx)
    out = _fin(acc2, den2, b2r)
    return out.reshape(_N, 1, _D)


# speedup vs baseline: 2.8516x; 1.2144x over previous
import functools

import jax
import jax.numpy as jnp
from jax import lax
from jax.experimental import pallas as pl
from jax.experimental.pallas import tpu as pltpu
from jax.experimental.pallas import tpu_sc as plsc

_N = 10000
_E = 320000
_D = 128

_NC = 2
_NS = 16
_NW = _NC * _NS
_EPW = _E // _NW
_C = 80
_NCH = _EPW // _C
_RPT = _N // _NS

_ZR = 125
_SUP = 5


def _edge_body(ft_hbm, el_hbm, er_hbm, idx_hbm,
               acc_out, den_out,
               idxb, el_v, er_v, rows_v, w_v, zden,
               acc_sh, den_sh, sem_g, sem_s, sem_i):
    cid = lax.axis_index("c")
    sid = lax.axis_index("s")
    wid = cid * _NS + sid

    pltpu.sync_copy(el_hbm, el_v)
    pltpu.sync_copy(er_hbm, er_v)

    zv = jnp.zeros((16,), jnp.float32)

    def zrow(i, _):
        for k in range(_D // 16):
            rows_v[0, i, pl.ds(k * 16, 16)] = zv
        return _
    lax.fori_loop(0, _C, zrow, None)

    def zden_row(i, _):
        zden[pl.ds(i * 16, 16)] = zv
        return _
    lax.fori_loop(0, 40, zden_row, None)

    for j in range(7):
        pltpu.sync_copy(rows_v.at[0],
                        acc_sh.at[pl.ds(sid * _RPT + j * _C, _C)])
    pltpu.sync_copy(rows_v.at[0, pl.ds(0, _RPT - 7 * _C)],
                    acc_sh.at[pl.ds(sid * _RPT + 7 * _C, _RPT - 7 * _C)])

    @pl.when(sid < _NS - 1)
    def _():
        pltpu.sync_copy(zden.at[pl.ds(0, 624)],
                        den_sh.at[pl.ds(sid * 624, 624)])

    @pl.when(sid == _NS - 1)
    def _():
        pltpu.sync_copy(zden, den_sh.at[pl.ds(624 * (_NS - 1), 640)])

    plsc.subcore_barrier()

    def chunk_work(ci, rb, sb, j, drain):
        nrb = 1 - rb

        def grp(gi, _):
            s16 = idxb[sb, 0, j, pl.ds(gi * 16, 16)]
            d16 = idxb[sb, 1, j, pl.ds(gi * 16, 16)]
            e = plsc.load_gather(el_v, [s16]) + plsc.load_gather(er_v, [d16])
            e = jnp.where(e >= 0.0, e, e * 0.2)
            w_v[rb, pl.ds(gi * 16, 16)] = jnp.exp(e)
            return _
        lax.fori_loop(0, _C // 16, grp, None)

        if drain:
            pltpu.make_async_copy(ft_hbm.at[pl.ds(0, _C)], rows_v.at[nrb],
                                  sem_s).wait()
            pltpu.make_async_copy(el_hbm.at[pl.ds(0, _C)], w_v.at[nrb],
                                  sem_s).wait()

        if j == 1:
            @pl.when(ci + 4 < _NCH)
            def _():
                row = wid * _NCH + ci + 4
                pltpu.async_copy(idx_hbm.at[0, pl.ds(row, _SUP)],
                                 idxb.at[1 - sb, 0], sem_i)
                pltpu.async_copy(idx_hbm.at[1, pl.ds(row, _SUP)],
                                 idxb.at[1 - sb, 1], sem_i)

        if j < _SUP - 1:
            pltpu.async_copy(ft_hbm.at[idxb.at[sb, 0, j + 1]],
                             rows_v.at[nrb], sem_g)
        else:
            @pl.when(ci + 1 < _NCH)
            def _():
                pltpu.make_async_copy(idx_hbm.at[0, pl.ds(0, _SUP)],
                                      idxb.at[1 - sb, 0], sem_i).wait()
                pltpu.make_async_copy(idx_hbm.at[1, pl.ds(0, _SUP)],
                                      idxb.at[1 - sb, 1], sem_i).wait()
                pltpu.async_copy(ft_hbm.at[idxb.at[1 - sb, 0, 0]],
                                 rows_v.at[nrb], sem_g)

        pltpu.make_async_copy(ft_hbm.at[pl.ds(0, _C)], rows_v.at[rb],
                              sem_g).wait()

        _dn = lax.GatherDimensionNumbers(offset_dims=(),
                                         collapsed_slice_dims=(0,),
                                         start_index_map=(0,))

        def scale(gi, _):
            wg = w_v[rb, pl.ds(gi * 16, 16)]
            for jj in range(16):
                ei = gi * 16 + jj
                wb = lax.gather(wg, jnp.full((16, 1), jj, jnp.int32), _dn,
                                (1,),
                                mode=lax.GatherScatterMode.PROMISE_IN_BOUNDS)
                for k in range(_D // 16):
                    rows_v[rb, ei, pl.ds(k * 16, 16)] = (
                        rows_v[rb, ei, pl.ds(k * 16, 16)] * wb)
            return _
        lax.fori_loop(0, _C // 16, scale, None)

        pltpu.async_copy(rows_v.at[rb], acc_sh.at[idxb.at[sb, 1, j]], sem_s,
                         add=True)
        pltpu.async_copy(w_v.at[rb], den_sh.at[idxb.at[sb, 1, j]], sem_s,
                         add=True)

    pltpu.sync_copy(idx_hbm.at[0, pl.ds(wid * _NCH, _SUP)], idxb.at[0, 0])
    pltpu.sync_copy(idx_hbm.at[1, pl.ds(wid * _NCH, _SUP)], idxb.at[0, 1])
    pltpu.async_copy(ft_hbm.at[idxb.at[0, 0, 0]], rows_v.at[0], sem_g)

    for j in range(_SUP):
        chunk_work(j, j % 2, 0, j, drain=(j > 0))

    def pair(p, _):
        base0 = 5 * (1 + 2 * p)
        for j in range(_SUP):
            chunk_work(base0 + j, (1 + j) % 2, 1, j, drain=True)
        base1 = 5 * (2 + 2 * p)
        for j in range(_SUP):
            chunk_work(base1 + j, j % 2, 0, j, drain=True)
        return _
    lax.fori_loop(0, (_NCH // _SUP - 1) // 2, pair, None)

    pltpu.make_async_copy(ft_hbm.at[pl.ds(0, _C)], rows_v.at[0], sem_s).wait()
    pltpu.make_async_copy(el_hbm.at[pl.ds(0, _C)], w_v.at[0], sem_s).wait()

    plsc.subcore_barrier()

    pltpu.sync_copy(acc_sh.at[pl.ds(sid * _RPT, _RPT)],
                    acc_out.at[cid, pl.ds(sid * _RPT, _RPT)])

    @pl.when(sid == 0)
    def _():
        pltpu.sync_copy(den_sh, den_out.at[cid])


def _edge_call(ft, el, er, idx):
    mesh = plsc.VectorSubcoreMesh(core_axis_name="c", subcore_axis_name="s",
                                  num_cores=_NC, num_subcores=_NS)
    f = pl.kernel(
        _edge_body,
        out_type=(jax.ShapeDtypeStruct((_NC, _N, _D), jnp.float32),
                  jax.ShapeDtypeStruct((_NC, _N), jnp.float32)),
        mesh=mesh,
        scratch_types=[
            pltpu.VMEM((2, 2, _SUP, _C), jnp.int32),
            pltpu.VMEM((_N,), jnp.float32),
            pltpu.VMEM((_N,), jnp.float32),
            pltpu.VMEM((2, _C, _D), jnp.float32),
            pltpu.VMEM((2, _C), jnp.float32),
            pltpu.VMEM((640,), jnp.float32),
            pltpu.VMEM_SHARED((_N, _D), jnp.float32),
            pltpu.VMEM_SHARED((_N,), jnp.float32),
            pltpu.SemaphoreType.DMA,
            pltpu.SemaphoreType.DMA,
            pltpu.SemaphoreType.DMA,
        ],
        compiler_params=pltpu.CompilerParams(use_tc_tiling_on_sc=False,
                                             needs_layout_passes=False),
    )
    return f(ft, el, er, idx)


_B = 2000


def _prep1_body(x_ref, w_ref, al_ref, ar_ref, ft_ref, el_ref, er_ref):
    ft = jnp.dot(x_ref[...], w_ref[...], preferred_element_type=jnp.float32)
    ft_ref[...] = ft
    el_ref[...] = jnp.sum(ft * al_ref[...], axis=1, keepdims=True)
    er_ref[...] = jnp.sum(ft * ar_ref[...], axis=1, keepdims=True)


def _prep1(x, W, al, ar):
    return pl.pallas_call(
        _prep1_body,
        grid=(_N // _B,),
        in_specs=[
            pl.BlockSpec((_B, _D), lambda i: (i, 0)),
            pl.BlockSpec((_D, _D), lambda i: (0, 0)),
            pl.BlockSpec((1, _D), lambda i: (0, 0)),
            pl.BlockSpec((1, _D), lambda i: (0, 0)),
        ],
        out_specs=[
            pl.BlockSpec((_B, _D), lambda i: (i, 0)),
            pl.BlockSpec((_B, 1), lambda i: (i, 0)),
            pl.BlockSpec((_B, 1), lambda i: (i, 0)),
        ],
        out_shape=[
            jax.ShapeDtypeStruct((_N, _D), jnp.float32),
            jax.ShapeDtypeStruct((_N, 1), jnp.float32),
            jax.ShapeDtypeStruct((_N, 1), jnp.float32),
        ],
    )(x, W, al, ar)


def _prep2_body(acc_ref, den_ref, b_ref, w_ref, al_ref, ar_ref,
                ft_ref, el_ref, er_ref):
    a = acc_ref[0] + acc_ref[1]
    dn = den_ref[0] + den_ref[1]
    h = a / (dn + 1e-9) + b_ref[...]
    ft = jnp.dot(h, w_ref[...], preferred_element_type=jnp.float32)
    ft_ref[...] = ft
    el_ref[...] = jnp.sum(ft * al_ref[...], axis=1, keepdims=True)
    er_ref[...] = jnp.sum(ft * ar_ref[...], axis=1, keepdims=True)


def _prep2(acc, den, b, W, al, ar):
    den = den.reshape(2, _N, 1)
    return pl.pallas_call(
        _prep2_body,
        grid=(_N // _B,),
        in_specs=[
            pl.BlockSpec((2, _B, _D), lambda i: (0, i, 0)),
            pl.BlockSpec((2, _B, 1), lambda i: (0, i, 0)),
            pl.BlockSpec((1, _D), lambda i: (0, 0)),
            pl.BlockSpec((_D, _D), lambda i: (0, 0)),
            pl.BlockSpec((1, _D), lambda i: (0, 0)),
            pl.BlockSpec((1, _D), lambda i: (0, 0)),
        ],
        out_specs=[
            pl.BlockSpec((_B, _D), lambda i: (i, 0)),
            pl.BlockSpec((_B, 1), lambda i: (i, 0)),
            pl.BlockSpec((_B, 1), lambda i: (i, 0)),
        ],
        out_shape=[
            jax.ShapeDtypeStruct((_N, _D), jnp.float32),
            jax.ShapeDtypeStruct((_N, 1), jnp.float32),
            jax.ShapeDtypeStruct((_N, 1), jnp.float32),
        ],
    )(acc, den, b, W, al, ar)


def _fin_body(acc_ref, den_ref, b_ref, out_ref):
    a = acc_ref[0] + acc_ref[1]
    dn = den_ref[0] + den_ref[1]
    out_ref[...] = a / (dn + 1e-9) + b_ref[...]


def _fin(acc, den, b):
    den = den.reshape(2, _N, 1)
    return pl.pallas_call(
        _fin_body,
        grid=(_N // _B,),
        in_specs=[
            pl.BlockSpec((2, _B, _D), lambda i: (0, i, 0)),
            pl.BlockSpec((2, _B, 1), lambda i: (0, i, 0)),
            pl.BlockSpec((1, _D), lambda i: (0, 0)),
        ],
        out_specs=pl.BlockSpec((_B, _D), lambda i: (i, 0)),
        out_shape=jax.ShapeDtypeStruct((_N, _D), jnp.float32),
    )(acc, den, b)


def kernel(g, in_feat, W1, al1, ar1, b1, W2, al2, ar2, b2):
    g = g.astype(jnp.int32)
    idx = g.reshape(2, _NW * _NCH, _C)
    b1r = b1.reshape(1, _D)
    b2r = b2.reshape(1, _D)

    ft1, el1, er1 = _prep1(in_feat, W1, al1, ar1)
    acc1, den1 = _edge_call(ft1, el1.reshape(_N), er1.reshape(_N), idx)
    ft2, el2, er2 = _prep2(acc1, den1, b1r, W2, al2, ar2)
    acc2, den2 = _edge_call(ft2, el2.reshape(_N), er2.reshape(_N), idx)
    out = _fin(acc2, den2, b2r)
    return out.reshape(_N, 1, _D)


# speedup vs baseline: 2.9282x; 1.0268x over previous
import functools

import jax
import jax.numpy as jnp
from jax import lax
from jax.experimental import pallas as pl
from jax.experimental.pallas import tpu as pltpu
from jax.experimental.pallas import tpu_sc as plsc

_N = 10000
_E = 320000
_D = 128

_NC = 2
_NS = 16
_NW = _NC * _NS
_EPW = _E // _NW
_C = 80
_NCH = _EPW // _C
_RPT = _N // _NS

_ZR = 125
_SUP = 5


def _edge_body(ft_hbm, el_hbm, er_hbm, idx_hbm,
               acc_out, den_out,
               idxb, el_v, er_v, rows_v, w_v, zden,
               acc_sh, den_sh, sem_g, sem_s, sem_i):
    cid = lax.axis_index("c")
    sid = lax.axis_index("s")
    wid = cid * _NS + sid

    pltpu.async_copy(idx_hbm.at[0, pl.ds(wid * _NCH, _SUP)], idxb.at[0, 0],
                     sem_i)
    pltpu.async_copy(idx_hbm.at[1, pl.ds(wid * _NCH, _SUP)], idxb.at[0, 1],
                     sem_i)
    pltpu.async_copy(el_hbm, el_v, sem_s)
    pltpu.async_copy(er_hbm, er_v, sem_s)

    zv = jnp.zeros((16,), jnp.float32)

    def zrow(i, _):
        for k in range(_D // 16):
            rows_v[1, i, pl.ds(k * 16, 16)] = zv
        return _
    lax.fori_loop(0, _C, zrow, None)

    def zden_row(i, _):
        zden[pl.ds(i * 16, 16)] = zv
        return _
    lax.fori_loop(0, 40, zden_row, None)

    pltpu.make_async_copy(idx_hbm.at[0, pl.ds(0, _SUP)], idxb.at[0, 0],
                          sem_i).wait()
    pltpu.make_async_copy(idx_hbm.at[1, pl.ds(0, _SUP)], idxb.at[0, 1],
                          sem_i).wait()
    pltpu.async_copy(ft_hbm.at[idxb.at[0, 0, 0]], rows_v.at[0], sem_g)

    for j in range(7):
        pltpu.async_copy(rows_v.at[1],
                         acc_sh.at[pl.ds(sid * _RPT + j * _C, _C)], sem_i)
    pltpu.async_copy(rows_v.at[1, pl.ds(0, _RPT - 7 * _C)],
                     acc_sh.at[pl.ds(sid * _RPT + 7 * _C, _RPT - 7 * _C)],
                     sem_i)

    @pl.when(sid < _NS - 1)
    def _():
        pltpu.async_copy(zden.at[pl.ds(0, 624)],
                         den_sh.at[pl.ds(sid * 624, 624)], sem_i)

    @pl.when(sid == _NS - 1)
    def _():
        pltpu.async_copy(zden, den_sh.at[pl.ds(624 * (_NS - 1), 640)], sem_i)

    for j in range(7):
        pltpu.make_async_copy(rows_v.at[1],
                              acc_sh.at[pl.ds(0, _C)], sem_i).wait()
    pltpu.make_async_copy(rows_v.at[1, pl.ds(0, _RPT - 7 * _C)],
                          acc_sh.at[pl.ds(0, _RPT - 7 * _C)], sem_i).wait()

    @pl.when(sid < _NS - 1)
    def _():
        pltpu.make_async_copy(zden.at[pl.ds(0, 624)],
                              den_sh.at[pl.ds(0, 624)], sem_i).wait()

    @pl.when(sid == _NS - 1)
    def _():
        pltpu.make_async_copy(zden, den_sh.at[pl.ds(0, 640)], sem_i).wait()

    pltpu.make_async_copy(el_hbm, el_v, sem_s).wait()
    pltpu.make_async_copy(er_hbm, er_v, sem_s).wait()

    plsc.subcore_barrier()

    def chunk_work(ci, rb, sb, j, drain):
        nrb = 1 - rb

        def grp(gi, _):
            s16 = idxb[sb, 0, j, pl.ds(gi * 16, 16)]
            d16 = idxb[sb, 1, j, pl.ds(gi * 16, 16)]
            e = plsc.load_gather(el_v, [s16]) + plsc.load_gather(er_v, [d16])
            e = jnp.where(e >= 0.0, e, e * 0.2)
            w_v[rb, pl.ds(gi * 16, 16)] = jnp.exp(e)
            return _
        lax.fori_loop(0, _C // 16, grp, None)

        if drain:
            pltpu.make_async_copy(ft_hbm.at[pl.ds(0, _C)], rows_v.at[nrb],
                                  sem_s).wait()
            pltpu.make_async_copy(el_hbm.at[pl.ds(0, _C)], w_v.at[nrb],
                                  sem_s).wait()

        if j == 1:
            @pl.when(ci + 4 < _NCH)
            def _():
                row = wid * _NCH + ci + 4
                pltpu.async_copy(idx_hbm.at[0, pl.ds(row, _SUP)],
                                 idxb.at[1 - sb, 0], sem_i)
                pltpu.async_copy(idx_hbm.at[1, pl.ds(row, _SUP)],
                                 idxb.at[1 - sb, 1], sem_i)

        if j < _SUP - 1:
            pltpu.async_copy(ft_hbm.at[idxb.at[sb, 0, j + 1]],
                             rows_v.at[nrb], sem_g)
        else:
            @pl.when(ci + 1 < _NCH)
            def _():
                pltpu.make_async_copy(idx_hbm.at[0, pl.ds(0, _SUP)],
                                      idxb.at[1 - sb, 0], sem_i).wait()
                pltpu.make_async_copy(idx_hbm.at[1, pl.ds(0, _SUP)],
                                      idxb.at[1 - sb, 1], sem_i).wait()
                pltpu.async_copy(ft_hbm.at[idxb.at[1 - sb, 0, 0]],
                                 rows_v.at[nrb], sem_g)

        pltpu.make_async_copy(ft_hbm.at[pl.ds(0, _C)], rows_v.at[rb],
                              sem_g).wait()

        _dn = lax.GatherDimensionNumbers(offset_dims=(),
                                         collapsed_slice_dims=(0,),
                                         start_index_map=(0,))

        def scale(gi, _):
            wg = w_v[rb, pl.ds(gi * 16, 16)]
            for jj in range(16):
                ei = gi * 16 + jj
                wb = lax.gather(wg, jnp.full((16, 1), jj, jnp.int32), _dn,
                                (1,),
                                mode=lax.GatherScatterMode.PROMISE_IN_BOUNDS)
                for k in range(_D // 16):
                    rows_v[rb, ei, pl.ds(k * 16, 16)] = (
                        rows_v[rb, ei, pl.ds(k * 16, 16)] * wb)
            return _
        lax.fori_loop(0, _C // 16, scale, None)

        pltpu.async_copy(rows_v.at[rb], acc_sh.at[idxb.at[sb, 1, j]], sem_s,
                         add=True)
        pltpu.async_copy(w_v.at[rb], den_sh.at[idxb.at[sb, 1, j]], sem_s,
                         add=True)

    for j in range(_SUP):
        chunk_work(j, j % 2, 0, j, drain=(j > 0))

    def pair(p, _):
        base0 = 5 * (1 + 2 * p)
        for j in range(_SUP):
            chunk_work(base0 + j, (1 + j) % 2, 1, j, drain=True)
        base1 = 5 * (2 + 2 * p)
        for j in range(_SUP):
            chunk_work(base1 + j, j % 2, 0, j, drain=True)
        return _
    lax.fori_loop(0, (_NCH // _SUP - 1) // 2, pair, None)

    pltpu.make_async_copy(ft_hbm.at[pl.ds(0, _C)], rows_v.at[0], sem_s).wait()
    pltpu.make_async_copy(el_hbm.at[pl.ds(0, _C)], w_v.at[0], sem_s).wait()

    plsc.subcore_barrier()

    pltpu.sync_copy(acc_sh.at[pl.ds(sid * _RPT, _RPT)],
                    acc_out.at[cid, pl.ds(sid * _RPT, _RPT)])

    @pl.when(sid == 0)
    def _():
        pltpu.sync_copy(den_sh, den_out.at[cid])


def _edge_call(ft, el, er, idx):
    mesh = plsc.VectorSubcoreMesh(core_axis_name="c", subcore_axis_name="s",
                                  num_cores=_NC, num_subcores=_NS)
    f = pl.kernel(
        _edge_body,
        out_type=(jax.ShapeDtypeStruct((_NC, _N, _D), jnp.float32),
                  jax.ShapeDtypeStruct((_NC, _N), jnp.float32)),
        mesh=mesh,
        scratch_types=[
            pltpu.VMEM((2, 2, _SUP, _C), jnp.int32),
            pltpu.VMEM((_N,), jnp.float32),
            pltpu.VMEM((_N,), jnp.float32),
            pltpu.VMEM((2, _C, _D), jnp.float32),
            pltpu.VMEM((2, _C), jnp.float32),
            pltpu.VMEM((640,), jnp.float32),
            pltpu.VMEM_SHARED((_N, _D), jnp.float32),
            pltpu.VMEM_SHARED((_N,), jnp.float32),
            pltpu.SemaphoreType.DMA,
            pltpu.SemaphoreType.DMA,
            pltpu.SemaphoreType.DMA,
        ],
        compiler_params=pltpu.CompilerParams(use_tc_tiling_on_sc=False,
                                             needs_layout_passes=False),
    )
    return f(ft, el, er, idx)


_B = 2000


def _prep1_body(x_ref, w_ref, al_ref, ar_ref, ft_ref, el_ref, er_ref):
    ft = jnp.dot(x_ref[...], w_ref[...], preferred_element_type=jnp.float32)
    ft_ref[...] = ft
    el_ref[...] = jnp.sum(ft * al_ref[...], axis=1, keepdims=True)
    er_ref[...] = jnp.sum(ft * ar_ref[...], axis=1, keepdims=True)


def _prep1(x, W, al, ar):
    return pl.pallas_call(
        _prep1_body,
        grid=(_N // _B,),
        in_specs=[
            pl.BlockSpec((_B, _D), lambda i: (i, 0)),
            pl.BlockSpec((_D, _D), lambda i: (0, 0)),
            pl.BlockSpec((1, _D), lambda i: (0, 0)),
            pl.BlockSpec((1, _D), lambda i: (0, 0)),
        ],
        out_specs=[
            pl.BlockSpec((_B, _D), lambda i: (i, 0)),
            pl.BlockSpec((_B, 1), lambda i: (i, 0)),
            pl.BlockSpec((_B, 1), lambda i: (i, 0)),
        ],
        out_shape=[
            jax.ShapeDtypeStruct((_N, _D), jnp.float32),
            jax.ShapeDtypeStruct((_N, 1), jnp.float32),
            jax.ShapeDtypeStruct((_N, 1), jnp.float32),
        ],
    )(x, W, al, ar)


def _prep2_body(acc_ref, den_ref, b_ref, w_ref, al_ref, ar_ref,
                ft_ref, el_ref, er_ref):
    a = acc_ref[0] + acc_ref[1]
    dn = den_ref[0] + den_ref[1]
    h = a / (dn + 1e-9) + b_ref[...]
    ft = jnp.dot(h, w_ref[...], preferred_element_type=jnp.float32)
    ft_ref[...] = ft
    el_ref[...] = jnp.sum(ft * al_ref[...], axis=1, keepdims=True)
    er_ref[...] = jnp.sum(ft * ar_ref[...], axis=1, keepdims=True)


def _prep2(acc, den, b, W, al, ar):
    den = den.reshape(2, _N, 1)
    return pl.pallas_call(
        _prep2_body,
        grid=(_N // _B,),
        in_specs=[
            pl.BlockSpec((2, _B, _D), lambda i: (0, i, 0)),
            pl.BlockSpec((2, _B, 1), lambda i: (0, i, 0)),
            pl.BlockSpec((1, _D), lambda i: (0, 0)),
            pl.BlockSpec((_D, _D), lambda i: (0, 0)),
            pl.BlockSpec((1, _D), lambda i: (0, 0)),
            pl.BlockSpec((1, _D), lambda i: (0, 0)),
        ],
        out_specs=[
            pl.BlockSpec((_B, _D), lambda i: (i, 0)),
            pl.BlockSpec((_B, 1), lambda i: (i, 0)),
            pl.BlockSpec((_B, 1), lambda i: (i, 0)),
        ],
        out_shape=[
            jax.ShapeDtypeStruct((_N, _D), jnp.float32),
            jax.ShapeDtypeStruct((_N, 1), jnp.float32),
            jax.ShapeDtypeStruct((_N, 1), jnp.float32),
        ],
    )(acc, den, b, W, al, ar)


def _fin_body(acc_ref, den_ref, b_ref, out_ref):
    a = acc_ref[0] + acc_ref[1]
    dn = den_ref[0] + den_ref[1]
    out_ref[...] = a / (dn + 1e-9) + b_ref[...]


def _fin(acc, den, b):
    den = den.reshape(2, _N, 1)
    return pl.pallas_call(
        _fin_body,
        grid=(_N // _B,),
        in_specs=[
            pl.BlockSpec((2, _B, _D), lambda i: (0, i, 0)),
            pl.BlockSpec((2, _B, 1), lambda i: (0, i, 0)),
            pl.BlockSpec((1, _D), lambda i: (0, 0)),
        ],
        out_specs=pl.BlockSpec((_B, _D), lambda i: (i, 0)),
        out_shape=jax.ShapeDtypeStruct((_N, _D), jnp.float32),
    )(acc, den, b)


def kernel(g, in_feat, W1, al1, ar1, b1, W2, al2, ar2, b2):
    g = g.astype(jnp.int32)
    idx = g.reshape(2, _NW * _NCH, _C)
    b1r = b1.reshape(1, _D)
    b2r = b2.reshape(1, _D)

    ft1, el1, er1 = _prep1(in_feat, W1, al1, ar1)
    acc1, den1 = _edge_call(ft1, el1.reshape(_N), er1.reshape(_N), idx)
    ft2, el2, er2 = _prep2(acc1, den1, b1r, W2, al2, ar2)
    acc2, den2 = _edge_call(ft2, el2.reshape(_N), er2.reshape(_N), idx)
    out = _fin(acc2, den2, b2r)
    return out.reshape(_N, 1, _D)
